# baseline (speedup 1.0000x reference)
import functools

import jax
import jax.numpy as jnp
from jax import lax
from jax.experimental import pallas as pl
from jax.experimental.pallas import tpu as pltpu
from jax.experimental.pallas import tpu_sc as plsc

N_NODES = 10000
N_EDGES = 320000
D = 128
D_BITS = 10
N_TYPE = 8
N_LAYERS = 3
N_GRAPHS = 10
N_PER = 1000
H_MLPD = 256

NC = 2
NS = 16
NW = NC * NS
LANES = 16

EPW = N_EDGES // NW
CHUNK = 64
NSLOT = 4
NJ = (EPW + CHUNK - 1) // CHUNK
EPW_PAD = NJ * CHUNK
ACC_ROWS = 10112
RPT = ACC_ROWS // NS
DEG_W = 16

def _wid():
    return lax.axis_index("s") * NC + lax.axis_index("c")


def _sc_deg_body(ei_hbm, zo_hbm, out_hbm, idx_e, ones_v, accd):
    cid = lax.axis_index("c")
    sid = lax.axis_index("s")
    wid = _wid()

    base = sid * RPT
    zfull, zrem = divmod(RPT, CHUNK)
    for k in range(zfull):
        pltpu.sync_copy(zo_hbm.at[0], accd.at[pl.ds(base + k * CHUNK, CHUNK)])
    if zrem:
        pltpu.sync_copy(zo_hbm.at[0, pl.ds(0, zrem)],
                        accd.at[pl.ds(base + zfull * CHUNK, zrem)])
    pltpu.sync_copy(zo_hbm.at[1], ones_v)
    pltpu.sync_copy(ei_hbm.at[wid], idx_e)
    plsc.subcore_barrier()

    def body(j, carry):
        pltpu.sync_copy(ones_v, accd.at[idx_e.at[j, 1]], add=True)
        return carry

    lax.fori_loop(0, NJ, body, 0)
    plsc.subcore_barrier()
    pltpu.sync_copy(accd.at[pl.ds(base, RPT)],
                    out_hbm.at[cid, pl.ds(base, RPT)])


def _sc_agg_body(hn_hbm, ei_hbm, zz_hbm, out_hbm, idxb, gbuf, acc, gsems,
                 isems):
    cid = lax.axis_index("c")
    sid = lax.axis_index("s")
    wid = _wid()

    base = sid * RPT
    zfull, zrem = divmod(RPT, CHUNK)
    for k in range(zfull):
        pltpu.sync_copy(zz_hbm, acc.at[pl.ds(base + k * CHUNK, CHUNK)])
    if zrem:
        pltpu.sync_copy(zz_hbm.at[pl.ds(0, zrem)],
                        acc.at[pl.ds(base + zfull * CHUNK, zrem)])
    plsc.subcore_barrier()

    pltpu.sync_copy(ei_hbm.at[wid, 0], idxb.at[0])
    pltpu.async_copy(hn_hbm.at[idxb.at[0, 0]], gbuf.at[0], gsems.at[0])
    for k in range(1, NSLOT):
        pltpu.async_copy(ei_hbm.at[wid, k], idxb.at[k], isems.at[k])
    for k in range(1, NSLOT - 1):
        pltpu.make_async_copy(ei_hbm.at[wid, k], idxb.at[k],
                              isems.at[k]).wait()
        pltpu.async_copy(hn_hbm.at[idxb.at[k, 0]], gbuf.at[k], gsems.at[k])

    def body(j, carry):
        slot = lax.rem(j, NSLOT)
        gnew = lax.rem(j + NSLOT - 1, NSLOT)
        pltpu.make_async_copy(hn_hbm.at[idxb.at[slot, 0]], gbuf.at[slot],
                              gsems.at[slot]).wait()

        @pl.when(j + NSLOT - 1 < NJ)
        def _():
            pltpu.make_async_copy(ei_hbm.at[wid, j + NSLOT - 1],
                                  idxb.at[gnew], isems.at[gnew]).wait()
            pltpu.async_copy(hn_hbm.at[idxb.at[gnew, 0]], gbuf.at[gnew],
                             gsems.at[gnew])

        pltpu.sync_copy(gbuf.at[slot], acc.at[idxb.at[slot, 1]], add=True)

        @pl.when(j + NSLOT < NJ)
        def _():
            pltpu.async_copy(ei_hbm.at[wid, j + NSLOT], idxb.at[slot],
                             isems.at[slot])

        return carry

    lax.fori_loop(0, NJ, body, 0)
    plsc.subcore_barrier()
    pltpu.sync_copy(acc.at[pl.ds(base, RPT)],
                    out_hbm.at[cid, pl.ds(base, RPT)])


@functools.lru_cache(maxsize=None)
def _sc_kernels():
    mesh = plsc.VectorSubcoreMesh(core_axis_name="c", subcore_axis_name="s",
                                  num_cores=NC, num_subcores=NS)
    sc_deg = pl.kernel(
        _sc_deg_body,
        out_type=jax.ShapeDtypeStruct((NC, ACC_ROWS, DEG_W), jnp.float32),
        mesh=mesh,
        scratch_types=[
            pltpu.VMEM((NJ, 2, CHUNK), jnp.int32),
            pltpu.VMEM((CHUNK, DEG_W), jnp.float32),
            pltpu.VMEM_SHARED((ACC_ROWS, DEG_W), jnp.float32),
        ],
    )
    sc_agg = pl.kernel(
        _sc_agg_body,
        out_type=jax.ShapeDtypeStruct((NC, ACC_ROWS, D), jnp.float32),
        mesh=mesh,
        scratch_types=[
            pltpu.VMEM((NSLOT, 2, CHUNK), jnp.int32),
            pltpu.VMEM((NSLOT, CHUNK, D), jnp.float32),
            pltpu.VMEM_SHARED((ACC_ROWS, D), jnp.float32),
            pltpu.SemaphoreType.DMA((NSLOT,)),
            pltpu.SemaphoreType.DMA((NSLOT,)),
        ],
    )
    return sc_deg, sc_agg


_BN = 1000
_GRID = N_NODES // _BN


def _ln(h, g, b):
    mu = jnp.mean(h, axis=-1, keepdims=True)
    var = jnp.mean((h - mu) * (h - mu), axis=-1, keepdims=True)
    return (h - mu) * lax.rsqrt(var + 1e-5) * g + b


def _tc_init_body(x_ref, nt_ref, bwt_ref, qe_ref, g_ref, b_ref,
                  h_ref, hn_ref):
    xb = x_ref[...]
    nt = nt_ref[...]
    oh = (lax.broadcasted_iota(jnp.int32, (_BN, N_TYPE), 1) == nt)
    oh = oh.astype(jnp.float32)
    h = (jnp.dot(xb, bwt_ref[...], preferred_element_type=jnp.float32)
         + jnp.dot(oh, qe_ref[...], preferred_element_type=jnp.float32))
    h_ref[...] = h
    hn_ref[...] = _ln(h, g_ref[...], b_ref[...])


def _tc_init(x, nt2, bwt, q_emb, g0, b0):
    return pl.pallas_call(
        _tc_init_body,
        grid=(_GRID,),
        in_specs=[
            pl.BlockSpec((_BN, D_BITS), lambda i: (i, 0)),
            pl.BlockSpec((_BN, 1), lambda i: (i, 0)),
            pl.BlockSpec((D_BITS, D), lambda i: (0, 0)),
            pl.BlockSpec((N_TYPE, D), lambda i: (0, 0)),
            pl.BlockSpec((1, D), lambda i: (0, 0)),
            pl.BlockSpec((1, D), lambda i: (0, 0)),
        ],
        out_specs=[
            pl.BlockSpec((_BN, D), lambda i: (i, 0)),
            pl.BlockSpec((_BN, D), lambda i: (i, 0)),
        ],
        out_shape=[
            jax.ShapeDtypeStruct((N_NODES, D), jnp.float32),
            jax.ShapeDtypeStruct((N_NODES, D), jnp.float32),
        ],
    )(x, nt2, bwt, q_emb, g0, b0)


def _tc_layer_body(h_ref, hn_ref, pp_ref, degp_ref, wlt_ref, bl_ref,
                   wrt_ref, g_ref, b_ref, ho_ref, hno_ref):
    deg = (degp_ref[0, :, :1] + degp_ref[1, :, :1])
    agg = ((pp_ref[0] + pp_ref[1]) / jnp.maximum(deg, 1.0))
    out = (jnp.dot(agg, wlt_ref[...], preferred_element_type=jnp.float32)
           + bl_ref[...]
           + jnp.dot(hn_ref[...], wrt_ref[...],
                     preferred_element_type=jnp.float32))
    h = jnp.maximum(out, 0.0) + h_ref[...]
    ho_ref[...] = h
    hno_ref[...] = _ln(h, g_ref[...], b_ref[...])


def _tc_layer(h, hn, pp, degp, wlt, bl, wrt, g, b):
    return pl.pallas_call(
        _tc_layer_body,
        grid=(_GRID,),
        in_specs=[
            pl.BlockSpec((_BN, D), lambda i: (i, 0)),
            pl.BlockSpec((_BN, D), lambda i: (i, 0)),
            pl.BlockSpec((NC, _BN, D), lambda i: (0, i, 0)),
            pl.BlockSpec((NC, _BN, DEG_W), lambda i: (0, i, 0)),
            pl.BlockSpec((D, D), lambda i: (0, 0)),
            pl.BlockSpec((1, D), lambda i: (0, 0)),
            pl.BlockSpec((D, D), lambda i: (0, 0)),
            pl.BlockSpec((1, D), lambda i: (0, 0)),
            pl.BlockSpec((1, D), lambda i: (0, 0)),
        ],
        out_specs=[
            pl.BlockSpec((_BN, D), lambda i: (i, 0)),
            pl.BlockSpec((_BN, D), lambda i: (i, 0)),
        ],
        out_shape=[
            jax.ShapeDtypeStruct((N_NODES, D), jnp.float32),
            jax.ShapeDtypeStruct((N_NODES, D), jnp.float32),
        ],
    )(h, hn, pp, degp, wlt, bl, wrt, g, b)


def _tc_readout_body(h_ref, bt_ref, mk_ref, m1w_ref, m1b_ref, m2w_ref, m2b_ref,
                     m3w_ref, m3b_ref, v1w_ref, v1b_ref, v2w_ref, v2b_ref,
                     v3w_ref, v3b_ref, z_ref, v_ref, hsum, cnt):
    i = pl.program_id(0)

    @pl.when(i == 0)
    def _():
        hsum[...] = jnp.zeros((N_GRAPHS, D), jnp.float32)
        cnt[...] = jnp.zeros((N_GRAPHS, D), jnp.float32)

    hb = h_ref[...]
    bt = bt_ref[...][0]
    m = (lax.broadcasted_iota(jnp.int32, (N_GRAPHS, _BN), 0) == bt)
    m = m.astype(jnp.float32)
    hsum[...] += jnp.dot(m, hb, preferred_element_type=jnp.float32)
    cnt[...] += jnp.broadcast_to(jnp.sum(m, axis=1, keepdims=True),
                                 (N_GRAPHS, D))

    a1 = jnp.maximum(jnp.dot(hb, m1w_ref[...],
                             preferred_element_type=jnp.float32)
                     + m1b_ref[...], 0.0)
    a2 = jnp.maximum(jnp.dot(a1, m2w_ref[...],
                             preferred_element_type=jnp.float32)
                     + m2b_ref[...], 0.0)
    zb = jnp.sum(a2 * m3w_ref[...], axis=1) + m3b_ref[0, 0]
    mk = mk_ref[...][0, 0]
    zb = jnp.where(mk != 0, zb, -jnp.inf)
    z_ref[...] = zb.reshape(1, 1, _BN)

    @pl.when(i == _GRID - 1)
    def _():
        means = hsum[...] / jnp.maximum(cnt[...], 1.0)
        v1 = jnp.maximum(jnp.dot(means, v1w_ref[...],
                                 preferred_element_type=jnp.float32)
                         + v1b_ref[...], 0.0)
        v2 = jnp.maximum(jnp.dot(v1, v2w_ref[...],
                                 preferred_element_type=jnp.float32)
                         + v2b_ref[...], 0.0)
        vv = jnp.sum(v2 * v3w_ref[...], axis=1, keepdims=True) + v3b_ref[0, 0]
        v_ref[...] = jnp.broadcast_to(vv, (N_GRAPHS, D))


def _tc_readout(h, bt3, mk3, m1w, m1b, m2w, m2b, m3w, m3b, v1w, v1b, v2w, v2b,
                v3w, v3b):
    const = lambda shape: pl.BlockSpec(shape, lambda i: tuple(0 for _ in shape))
    return pl.pallas_call(
        _tc_readout_body,
        grid=(_GRID,),
        in_specs=[
            pl.BlockSpec((_BN, D), lambda i: (i, 0)),
            pl.BlockSpec((1, 1, _BN), lambda i: (i, 0, 0)),
            pl.BlockSpec((1, 1, _BN), lambda i: (i, 0, 0)),
            const((D, H_MLPD)),
            const((1, H_MLPD)),
            const((H_MLPD, H_MLPD)),
            const((1, H_MLPD)),
            const((1, H_MLPD)),
            const((1, 1)),
            const((D, H_MLPD)),
            const((1, H_MLPD)),
            const((H_MLPD, H_MLPD)),
            const((1, H_MLPD)),
            const((1, H_MLPD)),
            const((1, 1)),
        ],
        out_specs=[
            pl.BlockSpec((1, 1, _BN), lambda i: (i, 0, 0)),
            pl.BlockSpec((N_GRAPHS, D), lambda i: (0, 0)),
        ],
        out_shape=[
            jax.ShapeDtypeStruct((_GRID, 1, _BN), jnp.float32),
            jax.ShapeDtypeStruct((N_GRAPHS, D), jnp.float32),
        ],
        scratch_shapes=[
            pltpu.VMEM((N_GRAPHS, D), jnp.float32),
            pltpu.VMEM((N_GRAPHS, D), jnp.float32),
        ],
    )(h, bt3, mk3, m1w, m1b, m2w, m2b, m3w, m3b, v1w, v1b, v2w, v2b, v3w, v3b)


def kernel(x, params, node_type, edge_index, action_mask, node_indices,
           batch, N):
    del node_indices, N
    f32, i32 = jnp.float32, jnp.int32
    src = edge_index[0].astype(i32)
    dst = edge_index[1].astype(i32)
    src_r = jnp.pad(src.reshape(NW, EPW),
                    ((0, 0), (0, EPW_PAD - EPW))).reshape(NW, NJ, CHUNK)
    dst_r = jnp.pad(dst.reshape(NW, EPW), ((0, 0), (0, EPW_PAD - EPW)),
                    constant_values=N_NODES).reshape(NW, NJ, CHUNK)
    ei_r = jnp.stack([src_r, dst_r], axis=2)

    _sc_deg, _sc_agg = _sc_kernels()
    zo = jnp.concatenate([jnp.zeros((1, CHUNK, DEG_W), f32),
                          jnp.ones((1, CHUNK, DEG_W), f32)])
    zz = jnp.zeros((CHUNK, D), f32)
    degp = _sc_deg(ei_r, zo)

    p = params
    h, hn = _tc_init(x.astype(f32), node_type.astype(i32)[:, None],
                     p['bW'].T, p['q_emb'],
                     p['ln_g'][0][None, :], p['ln_b'][0][None, :])

    for i in range(N_LAYERS):
        pp = _sc_agg(hn, ei_r, zz)
        g_next = p['ln_g'][(i + 1) % N_LAYERS][None, :]
        b_next = p['ln_b'][(i + 1) % N_LAYERS][None, :]
        h, hn = _tc_layer(h, hn, pp, degp,
                          p['Wl'][i].T, p['bl'][i][None, :], p['Wr'][i].T,
                          g_next, b_next)

    bt3 = batch.astype(i32).reshape(_GRID, 1, _BN)
    mk3 = action_mask.astype(i32).reshape(_GRID, 1, _BN)
    z3, vb = _tc_readout(h, bt3, mk3,
                         p['m1W'].T, p['m1b'][None, :],
                         p['m2W'].T, p['m2b'][None, :],
                         p['m3W'], p['m3b'][None, :],
                         p['v1W'].T, p['v1b'][None, :],
                         p['v2W'].T, p['v2b'][None, :],
                         p['v3W'], p['v3b'][None, :])
    z = z3.reshape(N_GRAPHS, N_PER)
    v = vb[:, :1]
    return (z, v)


# speedup vs baseline: 1.2630x; 1.2630x over previous
import functools

import jax
import jax.numpy as jnp
from jax import lax
from jax.experimental import pallas as pl
from jax.experimental.pallas import tpu as pltpu
from jax.experimental.pallas import tpu_sc as plsc

N_NODES = 10000
N_EDGES = 320000
D = 128
D_BITS = 10
N_TYPE = 8
N_LAYERS = 3
N_GRAPHS = 10
N_PER = 1000
H_MLPD = 256

NC = 2
NS = 16
NW = NC * NS
LANES = 16

EPW = N_EDGES // NW
CHUNK = 64
NSLOT = 4
NJ = (EPW + CHUNK - 1) // CHUNK
EPW_PAD = NJ * CHUNK
ACC_ROWS = 10112
RPT = ACC_ROWS // NS
DEG_W = 16

def _wid():
    return lax.axis_index("s") * NC + lax.axis_index("c")


def _sc_deg_body(ei_hbm, zo_hbm, out_hbm, idx_e, ones_v, accd, dsems):
    cid = lax.axis_index("c")
    sid = lax.axis_index("s")
    wid = _wid()

    base = sid * RPT
    zfull, zrem = divmod(RPT, CHUNK)
    for k in range(zfull):
        pltpu.sync_copy(zo_hbm.at[0], accd.at[pl.ds(base + k * CHUNK, CHUNK)])
    if zrem:
        pltpu.sync_copy(zo_hbm.at[0, pl.ds(0, zrem)],
                        accd.at[pl.ds(base + zfull * CHUNK, zrem)])
    pltpu.sync_copy(zo_hbm.at[1], ones_v)
    pltpu.sync_copy(ei_hbm.at[wid], idx_e)
    plsc.subcore_barrier()

    pltpu.async_copy(ones_v, accd.at[idx_e.at[0, 1]], dsems.at[0], add=True)

    def body(j, carry):
        s = lax.rem(j, 2)
        pltpu.async_copy(ones_v, accd.at[idx_e.at[j, 1]], dsems.at[s],
                         add=True)
        sp = lax.rem(j + 1, 2)
        pltpu.make_async_copy(ones_v, accd.at[idx_e.at[j - 1, 1]],
                              dsems.at[sp]).wait()
        return carry

    lax.fori_loop(1, NJ, body, 0)
    pltpu.make_async_copy(ones_v, accd.at[idx_e.at[NJ - 1, 1]],
                          dsems.at[(NJ - 1) % 2]).wait()
    plsc.subcore_barrier()
    pltpu.sync_copy(accd.at[pl.ds(base, RPT)],
                    out_hbm.at[cid, pl.ds(base, RPT)])


def _sc_agg_body(hn_hbm, ei_hbm, zz_hbm, out_hbm, idxb, gbuf, acc, gsems,
                 isems, ssems):
    cid = lax.axis_index("c")
    sid = lax.axis_index("s")
    wid = _wid()

    base = sid * RPT
    zfull, zrem = divmod(RPT, CHUNK)
    for k in range(zfull):
        pltpu.sync_copy(zz_hbm, acc.at[pl.ds(base + k * CHUNK, CHUNK)])
    if zrem:
        pltpu.sync_copy(zz_hbm.at[pl.ds(0, zrem)],
                        acc.at[pl.ds(base + zfull * CHUNK, zrem)])
    plsc.subcore_barrier()

    NI = 2 * NSLOT
    pltpu.sync_copy(ei_hbm.at[wid, 0], idxb.at[0])
    pltpu.async_copy(hn_hbm.at[idxb.at[0, 0]], gbuf.at[0], gsems.at[0])
    for k in range(1, NI):
        pltpu.async_copy(ei_hbm.at[wid, k], idxb.at[k], isems.at[k])
    for k in range(1, NSLOT - 1):
        pltpu.make_async_copy(ei_hbm.at[wid, k], idxb.at[k],
                              isems.at[k]).wait()
        pltpu.async_copy(hn_hbm.at[idxb.at[k, 0]], gbuf.at[k], gsems.at[k])

    def body(j, carry):
        slot = lax.rem(j, NSLOT)
        gnew = lax.rem(j + NSLOT - 1, NSLOT)
        islot = lax.rem(j, NI)
        inew = lax.rem(j + NSLOT - 1, NI)
        pltpu.make_async_copy(hn_hbm.at[idxb.at[islot, 0]], gbuf.at[slot],
                              gsems.at[slot]).wait()

        ifree = lax.rem(j + NI - 1, NI)

        @pl.when(j + NSLOT - 1 < NJ)
        def _():
            @pl.when(j > 0)
            def _():
                pltpu.make_async_copy(gbuf.at[gnew],
                                      acc.at[idxb.at[ifree, 1]],
                                      ssems.at[gnew]).wait()

            pltpu.make_async_copy(ei_hbm.at[wid, j + NSLOT - 1],
                                  idxb.at[inew], isems.at[inew]).wait()
            pltpu.async_copy(hn_hbm.at[idxb.at[inew, 0]], gbuf.at[gnew],
                             gsems.at[gnew])

            @pl.when(j + NI - 1 < NJ)
            def _():
                pltpu.async_copy(ei_hbm.at[wid, j + NI - 1], idxb.at[ifree],
                                 isems.at[ifree])

        pltpu.async_copy(gbuf.at[slot], acc.at[idxb.at[islot, 1]],
                         ssems.at[slot], add=True)
        return carry

    lax.fori_loop(0, NJ, body, 0)
    for s in range(min(NSLOT, NJ)):
        pltpu.make_async_copy(gbuf.at[s], acc.at[idxb.at[s, 1]],
                              ssems.at[s]).wait()
    plsc.subcore_barrier()
    pltpu.sync_copy(acc.at[pl.ds(base, RPT)],
                    out_hbm.at[cid, pl.ds(base, RPT)])


@functools.lru_cache(maxsize=None)
def _sc_kernels():
    mesh = plsc.VectorSubcoreMesh(core_axis_name="c", subcore_axis_name="s",
                                  num_cores=NC, num_subcores=NS)
    sc_deg = pl.kernel(
        _sc_deg_body,
        out_type=jax.ShapeDtypeStruct((NC, ACC_ROWS, DEG_W), jnp.float32),
        mesh=mesh,
        scratch_types=[
            pltpu.VMEM((NJ, 2, CHUNK), jnp.int32),
            pltpu.VMEM((CHUNK, DEG_W), jnp.float32),
            pltpu.VMEM_SHARED((ACC_ROWS, DEG_W), jnp.float32),
            pltpu.SemaphoreType.DMA((2,)),
        ],
    )
    sc_agg = pl.kernel(
        _sc_agg_body,
        out_type=jax.ShapeDtypeStruct((NC, ACC_ROWS, D), jnp.float32),
        mesh=mesh,
        scratch_types=[
            pltpu.VMEM((2 * NSLOT, 2, CHUNK), jnp.int32),
            pltpu.VMEM((NSLOT, CHUNK, D), jnp.float32),
            pltpu.VMEM_SHARED((ACC_ROWS, D), jnp.float32),
            pltpu.SemaphoreType.DMA((NSLOT,)),
            pltpu.SemaphoreType.DMA((2 * NSLOT,)),
            pltpu.SemaphoreType.DMA((NSLOT,)),
        ],
    )
    return sc_deg, sc_agg


_BN = 1000
_GRID = N_NODES // _BN


def _ln(h, g, b):
    mu = jnp.mean(h, axis=-1, keepdims=True)
    var = jnp.mean((h - mu) * (h - mu), axis=-1, keepdims=True)
    return (h - mu) * lax.rsqrt(var + 1e-5) * g + b


def _tc_init_body(x_ref, nt_ref, bwt_ref, qe_ref, g_ref, b_ref,
                  h_ref, hn_ref):
    xb = x_ref[...]
    nt = nt_ref[...]
    oh = (lax.broadcasted_iota(jnp.int32, (_BN, N_TYPE), 1) == nt)
    oh = oh.astype(jnp.float32)
    h = (jnp.dot(xb, bwt_ref[...], preferred_element_type=jnp.float32)
         + jnp.dot(oh, qe_ref[...], preferred_element_type=jnp.float32))
    h_ref[...] = h
    hn_ref[...] = _ln(h, g_ref[...], b_ref[...])


def _tc_init(x, nt2, bwt, q_emb, g0, b0):
    return pl.pallas_call(
        _tc_init_body,
        grid=(_GRID,),
        in_specs=[
            pl.BlockSpec((_BN, D_BITS), lambda i: (i, 0)),
            pl.BlockSpec((_BN, 1), lambda i: (i, 0)),
            pl.BlockSpec((D_BITS, D), lambda i: (0, 0)),
            pl.BlockSpec((N_TYPE, D), lambda i: (0, 0)),
            pl.BlockSpec((1, D), lambda i: (0, 0)),
            pl.BlockSpec((1, D), lambda i: (0, 0)),
        ],
        out_specs=[
            pl.BlockSpec((_BN, D), lambda i: (i, 0)),
            pl.BlockSpec((_BN, D), lambda i: (i, 0)),
        ],
        out_shape=[
            jax.ShapeDtypeStruct((N_NODES, D), jnp.float32),
            jax.ShapeDtypeStruct((N_NODES, D), jnp.float32),
        ],
    )(x, nt2, bwt, q_emb, g0, b0)


def _tc_layer_body(h_ref, hn_ref, pp_ref, degp_ref, wlt_ref, bl_ref,
                   wrt_ref, g_ref, b_ref, ho_ref, hno_ref):
    deg = (degp_ref[0, :, :1] + degp_ref[1, :, :1])
    agg = ((pp_ref[0] + pp_ref[1]) / jnp.maximum(deg, 1.0))
    out = (jnp.dot(agg, wlt_ref[...], preferred_element_type=jnp.float32)
           + bl_ref[...]
           + jnp.dot(hn_ref[...], wrt_ref[...],
                     preferred_element_type=jnp.float32))
    h = jnp.maximum(out, 0.0) + h_ref[...]
    ho_ref[...] = h
    hno_ref[...] = _ln(h, g_ref[...], b_ref[...])


def _tc_layer(h, hn, pp, degp, wlt, bl, wrt, g, b):
    return pl.pallas_call(
        _tc_layer_body,
        grid=(_GRID,),
        in_specs=[
            pl.BlockSpec((_BN, D), lambda i: (i, 0)),
            pl.BlockSpec((_BN, D), lambda i: (i, 0)),
            pl.BlockSpec((NC, _BN, D), lambda i: (0, i, 0)),
            pl.BlockSpec((NC, _BN, DEG_W), lambda i: (0, i, 0)),
            pl.BlockSpec((D, D), lambda i: (0, 0)),
            pl.BlockSpec((1, D), lambda i: (0, 0)),
            pl.BlockSpec((D, D), lambda i: (0, 0)),
            pl.BlockSpec((1, D), lambda i: (0, 0)),
            pl.BlockSpec((1, D), lambda i: (0, 0)),
        ],
        out_specs=[
            pl.BlockSpec((_BN, D), lambda i: (i, 0)),
            pl.BlockSpec((_BN, D), lambda i: (i, 0)),
        ],
        out_shape=[
            jax.ShapeDtypeStruct((N_NODES, D), jnp.float32),
            jax.ShapeDtypeStruct((N_NODES, D), jnp.float32),
        ],
    )(h, hn, pp, degp, wlt, bl, wrt, g, b)


def _tc_readout_body(h_ref, bt_ref, mk_ref, m1w_ref, m1b_ref, m2w_ref, m2b_ref,
                     m3w_ref, m3b_ref, v1w_ref, v1b_ref, v2w_ref, v2b_ref,
                     v3w_ref, v3b_ref, z_ref, v_ref, hsum, cnt):
    i = pl.program_id(0)

    @pl.when(i == 0)
    def _():
        hsum[...] = jnp.zeros((N_GRAPHS, D), jnp.float32)
        cnt[...] = jnp.zeros((N_GRAPHS, D), jnp.float32)

    hb = h_ref[...]
    bt = bt_ref[...][0]
    m = (lax.broadcasted_iota(jnp.int32, (N_GRAPHS, _BN), 0) == bt)
    m = m.astype(jnp.float32)
    hsum[...] += jnp.dot(m, hb, preferred_element_type=jnp.float32)
    cnt[...] += jnp.broadcast_to(jnp.sum(m, axis=1, keepdims=True),
                                 (N_GRAPHS, D))

    a1 = jnp.maximum(jnp.dot(hb, m1w_ref[...],
                             preferred_element_type=jnp.float32)
                     + m1b_ref[...], 0.0)
    a2 = jnp.maximum(jnp.dot(a1, m2w_ref[...],
                             preferred_element_type=jnp.float32)
                     + m2b_ref[...], 0.0)
    zb = jnp.sum(a2 * m3w_ref[...], axis=1) + m3b_ref[0, 0]
    mk = mk_ref[...][0, 0]
    zb = jnp.where(mk != 0, zb, -jnp.inf)
    z_ref[...] = zb.reshape(1, 1, _BN)

    @pl.when(i == _GRID - 1)
    def _():
        means = hsum[...] / jnp.maximum(cnt[...], 1.0)
        v1 = jnp.maximum(jnp.dot(means, v1w_ref[...],
                                 preferred_element_type=jnp.float32)
                         + v1b_ref[...], 0.0)
        v2 = jnp.maximum(jnp.dot(v1, v2w_ref[...],
                                 preferred_element_type=jnp.float32)
                         + v2b_ref[...], 0.0)
        vv = jnp.sum(v2 * v3w_ref[...], axis=1, keepdims=True) + v3b_ref[0, 0]
        v_ref[...] = jnp.broadcast_to(vv, (N_GRAPHS, D))


def _tc_readout(h, bt3, mk3, m1w, m1b, m2w, m2b, m3w, m3b, v1w, v1b, v2w, v2b,
                v3w, v3b):
    const = lambda shape: pl.BlockSpec(shape, lambda i: tuple(0 for _ in shape))
    return pl.pallas_call(
        _tc_readout_body,
        grid=(_GRID,),
        in_specs=[
            pl.BlockSpec((_BN, D), lambda i: (i, 0)),
            pl.BlockSpec((1, 1, _BN), lambda i: (i, 0, 0)),
            pl.BlockSpec((1, 1, _BN), lambda i: (i, 0, 0)),
            const((D, H_MLPD)),
            const((1, H_MLPD)),
            const((H_MLPD, H_MLPD)),
            const((1, H_MLPD)),
            const((1, H_MLPD)),
            const((1, 1)),
            const((D, H_MLPD)),
            const((1, H_MLPD)),
            const((H_MLPD, H_MLPD)),
            const((1, H_MLPD)),
            const((1, H_MLPD)),
            const((1, 1)),
        ],
        out_specs=[
            pl.BlockSpec((1, 1, _BN), lambda i: (i, 0, 0)),
            pl.BlockSpec((N_GRAPHS, D), lambda i: (0, 0)),
        ],
        out_shape=[
            jax.ShapeDtypeStruct((_GRID, 1, _BN), jnp.float32),
            jax.ShapeDtypeStruct((N_GRAPHS, D), jnp.float32),
        ],
        scratch_shapes=[
            pltpu.VMEM((N_GRAPHS, D), jnp.float32),
            pltpu.VMEM((N_GRAPHS, D), jnp.float32),
        ],
    )(h, bt3, mk3, m1w, m1b, m2w, m2b, m3w, m3b, v1w, v1b, v2w, v2b, v3w, v3b)


def kernel(x, params, node_type, edge_index, action_mask, node_indices,
           batch, N):
    del node_indices, N
    f32, i32 = jnp.float32, jnp.int32
    src = edge_index[0].astype(i32)
    dst = edge_index[1].astype(i32)
    src_r = jnp.pad(src.reshape(NW, EPW),
                    ((0, 0), (0, EPW_PAD - EPW))).reshape(NW, NJ, CHUNK)
    dst_r = jnp.pad(dst.reshape(NW, EPW), ((0, 0), (0, EPW_PAD - EPW)),
                    constant_values=N_NODES).reshape(NW, NJ, CHUNK)
    ei_r = jnp.stack([src_r, dst_r], axis=2)

    _sc_deg, _sc_agg = _sc_kernels()
    zo = jnp.concatenate([jnp.zeros((1, CHUNK, DEG_W), f32),
                          jnp.ones((1, CHUNK, DEG_W), f32)])
    zz = jnp.zeros((CHUNK, D), f32)
    degp = _sc_deg(ei_r, zo)

    p = params
    h, hn = _tc_init(x.astype(f32), node_type.astype(i32)[:, None],
                     p['bW'].T, p['q_emb'],
                     p['ln_g'][0][None, :], p['ln_b'][0][None, :])

    for i in range(N_LAYERS):
        pp = _sc_agg(hn, ei_r, zz)
        g_next = p['ln_g'][(i + 1) % N_LAYERS][None, :]
        b_next = p['ln_b'][(i + 1) % N_LAYERS][None, :]
        h, hn = _tc_layer(h, hn, pp, degp,
                          p['Wl'][i].T, p['bl'][i][None, :], p['Wr'][i].T,
                          g_next, b_next)

    bt3 = batch.astype(i32).reshape(_GRID, 1, _BN)
    mk3 = action_mask.astype(i32).reshape(_GRID, 1, _BN)
    z3, vb = _tc_readout(h, bt3, mk3,
                         p['m1W'].T, p['m1b'][None, :],
                         p['m2W'].T, p['m2b'][None, :],
                         p['m3W'], p['m3b'][None, :],
                         p['v1W'].T, p['v1b'][None, :],
                         p['v2W'].T, p['v2b'][None, :],
                         p['v3W'], p['v3b'][None, :])
    z = z3.reshape(N_GRAPHS, N_PER)
    v = vb[:, :1]
    return (z, v)
